# baseline (speedup 1.0000x reference)
import functools
import math

import jax
import jax.numpy as jnp
from jax import lax
from jax.experimental import pallas as pl
from jax.experimental.pallas import tpu as pltpu
from jax.experimental.pallas import tpu_sc as plsc

D_MODEL = 64
SCALE = math.sqrt(D_MODEL)

NUM_CORES = 2
NUM_SUBCORES = 16
NUM_WORKERS = NUM_CORES * NUM_SUBCORES
LANES = 16

IDX_PER_STREAM = 128
STREAMS_PER_CHUNK = 2
CHUNK = IDX_PER_STREAM * STREAMS_PER_CHUNK
BB = CHUNK // 128
MINI_PITCH = 17


@functools.partial(jax.jit, static_argnums=(2, 3, 4))
def _emb_lookup(xs_flat, table_lin, n_b, n_s, d_model):
    chunks_per_s = n_b // CHUNK
    assert chunks_per_s == 16 and d_model == 64
    s_base, extra = divmod(n_s, NUM_WORKERS)
    mesh = plsc.VectorSubcoreMesh(core_axis_name="c", subcore_axis_name="s")

    @functools.partial(
        pl.kernel,
        out_type=jax.ShapeDtypeStruct(
            (n_s, 8, n_b // 128, 8, 128), jnp.float32),
        mesh=mesh,
        scratch_types=[
            pltpu.VMEM((2, n_b), jnp.int32),
            pltpu.VMEM((2, CHUNK, d_model), jnp.float32),
            pltpu.VMEM((2, 8, BB, 8, 128), jnp.float32),
            pltpu.VMEM((CHUNK // LANES, 4 * LANES * MINI_PITCH),
                       jnp.float32),
            pltpu.SemaphoreType.DMA,
            pltpu.SemaphoreType.DMA,
            pltpu.SemaphoreType.DMA,
            pltpu.SemaphoreType.DMA,
        ],
        compiler_params=pltpu.CompilerParams(use_tc_tiling_on_sc=False,
                                             needs_layout_passes=False),
    )
    def emb_kernel(x_hbm, tab_hbm, out_hbm, idx_v, rows_v, trans_v, mini_v,
                   gsem0, gsem1, wsem0, wsem1):
        wid = lax.axis_index("s") * NUM_CORES + lax.axis_index("c")
        s_lo = wid * s_base + jnp.minimum(wid, extra)
        s_hi = s_lo + s_base + jnp.where(wid < extra, 1, 0)
        u_lo = s_lo * chunks_per_s
        n_pairs = (s_hi - s_lo) * (chunks_per_s // 2)
        gsems = (gsem0, gsem1)
        wsems = (wsem0, wsem1)
        iota16 = lax.iota(jnp.int32, 16)
        diag = [iota16 * MINI_PITCH + r for r in range(LANES)]

        def stage_idx(s):
            pltpu.sync_copy(x_hbm.at[pl.ds(s * n_b, n_b)],
                            idx_v.at[s % 2])

        def fire_gathers(u, buf):
            s = u // chunks_per_s
            c = u % chunks_per_s
            for j in range(STREAMS_PER_CHUNK):
                pltpu.async_copy(
                    tab_hbm.at[idx_v.at[s % 2, pl.ds(
                        c * CHUNK + j * IDX_PER_STREAM, IDX_PER_STREAM)]],
                    rows_v.at[buf, pl.ds(j * IDX_PER_STREAM, IDX_PER_STREAM)],
                    gsems[buf],
                )

        def drain_gathers(buf):
            pltpu.make_async_copy(
                tab_hbm.at[pl.ds(0, CHUNK)], rows_v.at[buf], gsems[buf],
            ).wait()

        def transpose_scale(buf):
            def g_body(g, carry):
                r0 = g * LANES
                bbv = g // 8
                bi0 = (g % 8) * LANES
                n_dg = d_model // LANES
                for dg in range(n_dg + 1):
                    if dg < n_dg:
                        base = dg * LANES * MINI_PITCH
                        d0 = dg * LANES
                        vals = [rows_v[buf, r0 + r, pl.ds(d0, LANES)]
                                for r in range(LANES)]
                        for r in range(LANES):
                            plsc.store_scatter(mini_v.at[g],
                                               [diag[r] + base], vals[r])
                    if dg >= 1:
                        pbase = (dg - 1) * LANES * MINI_PITCH
                        pd0 = (dg - 1) * LANES
                        cols = [mini_v[g, pl.ds(pbase + c * MINI_PITCH,
                                                LANES)]
                                for c in range(LANES)]
                        for c in range(LANES):
                            d = pd0 + c
                            trans_v[buf, d // 8, bbv, d % 8,
                                    pl.ds(bi0, LANES)] = cols[c] * SCALE
                return carry

            lax.fori_loop(0, CHUNK // LANES, g_body, 0, unroll=False)

        def wb_desc(u, buf):
            s = u // chunks_per_s
            c = u % chunks_per_s
            return pltpu.make_async_copy(
                trans_v.at[buf],
                out_hbm.at[s, :, pl.ds(c * BB, BB)],
                wsems[buf],
            )

        stage_idx(s_lo)
        fire_gathers(u_lo, 0)

        def pair_body(p, carry):
            u0 = u_lo + 2 * p
            fire_gathers(u0 + 1, 1)
            drain_gathers(0)

            @pl.when(p >= 1)
            def _():
                wb_desc(u0 - 2, 0).wait()
            transpose_scale(0)
            wb_desc(u0, 0).start()

            @pl.when(p < n_pairs - 1)
            def _():
                u2 = u0 + 2

                @pl.when(u2 % chunks_per_s == 0)
                def _():
                    stage_idx(u2 // chunks_per_s)
                fire_gathers(u2, 0)
            drain_gathers(1)

            @pl.when(p >= 1)
            def _():
                wb_desc(u0 - 1, 1).wait()
            transpose_scale(1)
            wb_desc(u0 + 1, 1).start()
            return carry

        lax.fori_loop(0, n_pairs, pair_body, 0, unroll=False)
        u_last = u_lo + 2 * n_pairs - 1
        wb_desc(u_last - 1, 0).wait()
        wb_desc(u_last, 1).wait()

    return emb_kernel(xs_flat, table_lin)


def kernel(x, table):
    n_b, n_s = x.shape
    n_v, d_model = table.shape
    xs_flat = x.T.reshape(n_s * n_b) * 2
    tpad = jnp.pad(table, ((0, 0), (0, 128 - d_model)))
    table_lin = tpad.reshape(2 * n_v, d_model)
    out5 = _emb_lookup(xs_flat, table_lin, n_b, n_s, d_model)
    return out5.transpose(2, 4, 0, 1, 3).reshape(n_b, n_s, d_model)


# speedup vs baseline: 1.1334x; 1.1334x over previous
import functools
import math

import jax
import jax.numpy as jnp
from jax import lax
from jax.experimental import pallas as pl
from jax.experimental.pallas import tpu as pltpu
from jax.experimental.pallas import tpu_sc as plsc

D_MODEL = 64
SCALE = math.sqrt(D_MODEL)

NUM_CORES = 2
NUM_SUBCORES = 16
NUM_WORKERS = NUM_CORES * NUM_SUBCORES
LANES = 16

IDX_PER_STREAM = 128
STREAMS_PER_CHUNK = 2
CHUNK = IDX_PER_STREAM * STREAMS_PER_CHUNK
BB = CHUNK // 128
MINI_PITCH = 17


@functools.partial(jax.jit, static_argnums=(2, 3, 4))
def _emb_lookup(xs_flat, table_lin, n_b, n_s, d_model):
    chunks_per_s = n_b // CHUNK
    assert chunks_per_s == 16 and d_model == 64
    s_base, extra = divmod(n_s, NUM_WORKERS)
    mesh = plsc.VectorSubcoreMesh(core_axis_name="c", subcore_axis_name="s")

    @functools.partial(
        pl.kernel,
        out_type=jax.ShapeDtypeStruct(
            (n_s, 8, n_b // 128, 8, 128), jnp.float32),
        mesh=mesh,
        scratch_types=[
            pltpu.VMEM((2, n_b), jnp.int32),
            pltpu.VMEM((2, CHUNK, d_model), jnp.float32),
            pltpu.VMEM((2, 8, BB, 8, 128), jnp.float32),
            pltpu.VMEM((CHUNK // LANES, 4 * LANES * MINI_PITCH),
                       jnp.float32),
            pltpu.SemaphoreType.DMA,
            pltpu.SemaphoreType.DMA,
            pltpu.SemaphoreType.DMA,
            pltpu.SemaphoreType.DMA,
        ],
        compiler_params=pltpu.CompilerParams(use_tc_tiling_on_sc=False,
                                             needs_layout_passes=False),
    )
    def emb_kernel(x_hbm, tab_hbm, out_hbm, idx_v, rows_v, trans_v, mini_v,
                   gsem0, gsem1, wsem0, wsem1):
        wid = lax.axis_index("s") * NUM_CORES + lax.axis_index("c")
        s_lo = wid * s_base + jnp.minimum(wid, extra)
        s_hi = s_lo + s_base + jnp.where(wid < extra, 1, 0)
        u_lo = s_lo * chunks_per_s
        n_pairs = (s_hi - s_lo) * (chunks_per_s // 2)
        gsems = (gsem0, gsem1)
        wsems = (wsem0, wsem1)
        iota16 = lax.iota(jnp.int32, 16)
        diag = [iota16 * MINI_PITCH + r for r in range(LANES)]

        def stage_idx(s):
            pltpu.sync_copy(x_hbm.at[pl.ds(s * n_b, n_b)],
                            idx_v.at[s % 2])

        def fire_gathers(u, buf):
            s = u // chunks_per_s
            c = u % chunks_per_s
            for j in range(STREAMS_PER_CHUNK):
                pltpu.async_copy(
                    tab_hbm.at[idx_v.at[s % 2, pl.ds(
                        c * CHUNK + j * IDX_PER_STREAM, IDX_PER_STREAM)]],
                    rows_v.at[buf, pl.ds(j * IDX_PER_STREAM, IDX_PER_STREAM)],
                    gsems[buf],
                )

        def drain_gathers(buf):
            pltpu.make_async_copy(
                tab_hbm.at[pl.ds(0, CHUNK)], rows_v.at[buf], gsems[buf],
            ).wait()

        def transpose_scale(buf):
            def g_body(g, carry):
                r0 = g * LANES
                bbv = g // 8
                bi0 = (g % 8) * LANES
                def dg_body(dg, carry2):
                    base = dg * (LANES * MINI_PITCH)
                    d0 = dg * LANES
                    vals = [rows_v[buf, r0 + r, pl.ds(d0, LANES)]
                            for r in range(LANES)]
                    for r in range(LANES):
                        plsc.store_scatter(mini_v.at[g],
                                           [diag[r] + base], vals[r])
                    cols = [mini_v[g, pl.ds(base + c * MINI_PITCH, LANES)]
                            for c in range(LANES)]
                    for c in range(LANES):
                        dd = d0 + c
                        trans_v[buf, dd // 8, bbv, dd % 8,
                                pl.ds(bi0, LANES)] = cols[c] * SCALE
                    return carry2

                lax.fori_loop(0, d_model // LANES, dg_body, 0,
                              unroll=False)
                return carry

            lax.fori_loop(0, CHUNK // LANES, g_body, 0, unroll=False)

        def wb_desc(u, buf):
            s = u // chunks_per_s
            c = u % chunks_per_s
            return pltpu.make_async_copy(
                trans_v.at[buf],
                out_hbm.at[s, :, pl.ds(c * BB, BB)],
                wsems[buf],
            )

        stage_idx(s_lo)
        fire_gathers(u_lo, 0)

        def pair_body(p, carry):
            u0 = u_lo + 2 * p
            fire_gathers(u0 + 1, 1)
            drain_gathers(0)

            @pl.when(p >= 1)
            def _():
                wb_desc(u0 - 2, 0).wait()
            transpose_scale(0)
            wb_desc(u0, 0).start()

            @pl.when(p < n_pairs - 1)
            def _():
                u2 = u0 + 2

                @pl.when(u2 % chunks_per_s == 0)
                def _():
                    stage_idx(u2 // chunks_per_s)
                fire_gathers(u2, 0)
            drain_gathers(1)

            @pl.when(p >= 1)
            def _():
                wb_desc(u0 - 1, 1).wait()
            transpose_scale(1)
            wb_desc(u0 + 1, 1).start()
            return carry

        lax.fori_loop(0, n_pairs, pair_body, 0, unroll=False)
        u_last = u_lo + 2 * n_pairs - 1
        wb_desc(u_last - 1, 0).wait()
        wb_desc(u_last, 1).wait()

    return emb_kernel(xs_flat, table_lin)


def kernel(x, table):
    n_b, n_s = x.shape
    n_v, d_model = table.shape
    xs_flat = x.T.reshape(n_s * n_b) * 2
    tpad = jnp.pad(table, ((0, 0), (0, 128 - d_model)))
    table_lin = tpad.reshape(2 * n_v, d_model)
    out5 = _emb_lookup(xs_flat, table_lin, n_b, n_s, d_model)
    return out5.transpose(2, 4, 0, 1, 3).reshape(n_b, n_s, d_model)
